# baseline (speedup 1.0000x reference)
import functools

import jax
import jax.numpy as jnp
from jax import lax
from jax.experimental import pallas as pl
from jax.experimental.pallas import tpu as pltpu
from jax.experimental.pallas import tpu_sc as plsc

_NC = 2
_NS = 16
_NW = _NC * _NS
_L = 16

_B = 16384
_D = 64
_G = 8
_DIM0, _DIM1, _DIM2 = 100, 100, 100
_NSTATES = _DIM0 * _DIM1 * _DIM2
_BPW = _B // _NW
_NGRP = _BPW // _G


def _make_sc_call():
    mesh = plsc.VectorSubcoreMesh(core_axis_name="c", subcore_axis_name="s")

    @functools.partial(
        pl.kernel,
        mesh=mesh,
        out_type=jax.ShapeDtypeStruct((_G, _G, _B), jnp.float32),
        compiler_params=pltpu.CompilerParams(use_tc_tiling_on_sc=True,
                                             needs_layout_passes=False),
        scratch_types=[
            pltpu.VMEM((_BPW,), jnp.int32),
            pltpu.VMEM((_BPW,), jnp.int32),
            pltpu.VMEM((_BPW,), jnp.int32),
            pltpu.VMEM((_BPW,), jnp.int32),
            [pltpu.VMEM((_G, _G, 128), jnp.float32) for _ in range(4)],
            pltpu.VMEM((_G, _G, _BPW), jnp.float32),
            [pltpu.SemaphoreType.DMA for _ in range(4)],
            pltpu.SemaphoreType.DMA,
        ],
    )
    def tabular_gather(state_hbm, table_hbm, out_hbm,
                       s0_v, s1_v, s2_v, idx_v, banks, cols_v,
                       sems, osem):
        wid = lax.axis_index("s") * _NC + lax.axis_index("c")
        base = wid * _BPW

        pltpu.sync_copy(state_hbm.at[pl.ds(0 * _B + base, _BPW)], s0_v)
        pltpu.sync_copy(state_hbm.at[pl.ds(1 * _B + base, _BPW)], s1_v)
        pltpu.sync_copy(state_hbm.at[pl.ds(2 * _B + base, _BPW)], s2_v)

        for k in range(_BPW // _L):
            sl = pl.ds(k * _L, _L)
            a = jnp.minimum(jnp.maximum(s0_v[sl], 0), _DIM0 - 1)
            b = jnp.minimum(jnp.maximum(s1_v[sl], 0), _DIM1 - 1)
            c = jnp.minimum(jnp.maximum(s2_v[sl], 0), _DIM2 - 1)
            idx_v[sl] = a * (_DIM1 * _DIM2) + b * _DIM2 + c

        lanes = lax.iota(jnp.int32, _L)
        ghsel = []
        for q in range(_D // _L):
            j = lanes + jnp.int32(q * _L)
            ghsel.append((j >> 3, j & 7))

        def extract(k, j):
            vec = idx_v[pl.ds((k >> 1) * _L, _L)]
            jb = (k & 1) * _G
            return lax.reduce_max(jnp.where(lanes == jb + j, vec, 0),
                                  axes=(0,))

        def fetch(k, bank, sem):
            for j in range(_G):
                r = extract(k, j)
                r16 = pl.multiple_of((r >> 4) << 4, _L)
                pltpu.async_copy(
                    table_hbm.at[:, :, pl.ds(r16, _L)],
                    bank.at[:, :, pl.ds(j * _L, _L)], sem)

        def drain(bank, sem):
            for j in range(_G):
                pltpu.make_async_copy(
                    table_hbm.at[:, :, pl.ds(0, _L)],
                    bank.at[:, :, pl.ds(j * _L, _L)], sem).wait()

        def select(k, bank):
            for j in range(_G):
                r = extract(k, j)
                off = jnp.broadcast_to(j * _L + (r & 15), (_L,))
                i_bc = jnp.broadcast_to(k * _G + j, (_L,))
                for q in range(_D // _L):
                    gq, hq = ghsel[q]
                    vals = plsc.load_gather(bank, [gq, hq, off])
                    plsc.store_scatter(cols_v, [gq, hq, i_bc], vals)

        nb = len(banks)
        for p in range(nb - 1):
            fetch(p, banks[p], sems[p])

        @pl.loop(0, _NGRP // nb - 1)
        def _pipelined(t):
            for p in range(nb):
                k = nb * t + p
                fetch(k + nb - 1, banks[(p + nb - 1) % nb],
                      sems[(p + nb - 1) % nb])
                drain(banks[p], sems[p])
                select(k, banks[p])

        k0 = _NGRP - nb
        fetch(_NGRP - 1, banks[(nb - 1) % nb], sems[(nb - 1) % nb])
        for p in range(nb):
            drain(banks[p], sems[p])
            select(k0 + p, banks[p])

        outs = [
            pltpu.async_copy(cols_v.at[g], out_hbm.at[g, :, pl.ds(base, _BPW)],
                             osem)
            for g in range(_G)
        ]
        for o in outs:
            o.wait()

    return tabular_gather


_sc_call = _make_sc_call()


def kernel(state, params):
    flat = state.reshape(-1, state.shape[-1])
    state_t = flat.T.reshape(-1)
    table3 = params.T.reshape(_G, _G, _NSTATES)
    out3 = _sc_call(state_t, table3)
    return out3.reshape(_D, _B).T


# speedup vs baseline: 1.1425x; 1.1425x over previous
import functools

import jax
import jax.numpy as jnp
from jax import lax
from jax.experimental import pallas as pl
from jax.experimental.pallas import tpu as pltpu
from jax.experimental.pallas import tpu_sc as plsc

_NC = 2
_NS = 16
_NW = _NC * _NS
_L = 16

_B = 16384
_D = 64
_G = 8
_DIM0, _DIM1, _DIM2 = 100, 100, 100
_NSTATES = _DIM0 * _DIM1 * _DIM2
_BPW = _B // _NW
_NGRP = _BPW // _G


def _make_sc_call():
    mesh = plsc.VectorSubcoreMesh(core_axis_name="c", subcore_axis_name="s")

    @functools.partial(
        pl.kernel,
        mesh=mesh,
        out_type=jax.ShapeDtypeStruct((_G, _G, _B), jnp.float32),
        compiler_params=pltpu.CompilerParams(use_tc_tiling_on_sc=True,
                                             needs_layout_passes=False),
        scratch_types=[
            pltpu.VMEM((_BPW,), jnp.int32),
            pltpu.VMEM((_BPW,), jnp.int32),
            pltpu.VMEM((_BPW,), jnp.int32),
            pltpu.VMEM((_BPW,), jnp.int32),
            [pltpu.VMEM((_G, _G, 128), jnp.float32) for _ in range(2)],
            pltpu.VMEM((_G, _G, _BPW), jnp.float32),
            [pltpu.SemaphoreType.DMA for _ in range(2)],
            pltpu.SemaphoreType.DMA,
        ],
    )
    def tabular_gather(state_hbm, table_hbm, out_hbm,
                       s0_v, s1_v, s2_v, idx_v, banks, cols_v,
                       sems, osem):
        wid = lax.axis_index("s") * _NC + lax.axis_index("c")
        base = wid * _BPW

        pltpu.sync_copy(state_hbm.at[pl.ds(0 * _B + base, _BPW)], s0_v)
        pltpu.sync_copy(state_hbm.at[pl.ds(1 * _B + base, _BPW)], s1_v)
        pltpu.sync_copy(state_hbm.at[pl.ds(2 * _B + base, _BPW)], s2_v)

        for k in range(_BPW // _L):
            sl = pl.ds(k * _L, _L)
            a = jnp.minimum(jnp.maximum(s0_v[sl], 0), _DIM0 - 1)
            b = jnp.minimum(jnp.maximum(s1_v[sl], 0), _DIM1 - 1)
            c = jnp.minimum(jnp.maximum(s2_v[sl], 0), _DIM2 - 1)
            idx_v[sl] = a * (_DIM1 * _DIM2) + b * _DIM2 + c

        lanes = lax.iota(jnp.int32, _L)
        ghsel = []
        for q in range(_D // _L):
            j = lanes + jnp.int32(q * _L)
            ghsel.append((j >> 3, j & 7))

        def extract(k, j):
            vec = idx_v[pl.ds((k >> 1) * _L, _L)]
            jb = (k & 1) * _G
            return lax.reduce_max(jnp.where(lanes == jb + j, vec, 0),
                                  axes=(0,))

        def fetch(k, bank, sem):
            for j in range(_G):
                r = extract(k, j)
                r8 = pl.multiple_of((r >> 3) << 3, _G)
                pltpu.async_copy(
                    table_hbm.at[:, :, pl.ds(r8, _G)],
                    bank.at[:, :, pl.ds(j * _L, _G)], sem)

        def drain(bank, sem):
            for j in range(_G):
                pltpu.make_async_copy(
                    table_hbm.at[:, :, pl.ds(0, _G)],
                    bank.at[:, :, pl.ds(j * _L, _G)], sem).wait()

        def select(k, bank):
            for j in range(_G):
                r = extract(k, j)
                off = jnp.broadcast_to(j * _L + (r & 7), (_L,))
                i_bc = jnp.broadcast_to(k * _G + j, (_L,))
                for q in range(_D // _L):
                    gq, hq = ghsel[q]
                    vals = plsc.load_gather(bank, [gq, hq, off])
                    plsc.store_scatter(cols_v, [gq, hq, i_bc], vals)

        nb = len(banks)
        for p in range(nb - 1):
            fetch(p, banks[p], sems[p])

        @pl.loop(0, _NGRP // nb - 1)
        def _pipelined(t):
            for p in range(nb):
                k = nb * t + p
                fetch(k + nb - 1, banks[(p + nb - 1) % nb],
                      sems[(p + nb - 1) % nb])
                drain(banks[p], sems[p])
                select(k, banks[p])

        k0 = _NGRP - nb
        fetch(_NGRP - 1, banks[(nb - 1) % nb], sems[(nb - 1) % nb])
        for p in range(nb):
            drain(banks[p], sems[p])
            select(k0 + p, banks[p])

        outs = [
            pltpu.async_copy(cols_v.at[g], out_hbm.at[g, :, pl.ds(base, _BPW)],
                             osem)
            for g in range(_G)
        ]
        for o in outs:
            o.wait()

    return tabular_gather


_sc_call = _make_sc_call()


def kernel(state, params):
    flat = state.reshape(-1, state.shape[-1])
    state_t = flat.T.reshape(-1)
    table3 = params.T.reshape(_G, _G, _NSTATES)
    out3 = _sc_call(state_t, table3)
    return out3.reshape(_D, _B).T
